# baseline (speedup 1.0000x reference)
import jax
import jax.numpy as jnp
from jax.experimental import pallas as pl
from jax.experimental.pallas import tpu as pltpu

_NR = 8


def _body(xt_ref, emb_ref, wt_ref, b_ref, o_hbm, mscr, scratch, sems):
    B = xt_ref.shape[1]
    N = scratch.shape[0]
    R = N // _NR
    x0 = xt_ref[0:1, :]
    x1 = xt_ref[1:2, :]
    e0 = emb_ref[0, 0]
    e1 = emb_ref[0, 1]
    e = jnp.where(x1 >= 1.0, e1, e0)
    one = jnp.ones_like(x0)
    mscr[...] = jnp.concatenate([x0, e, one], axis=0)
    m = mscr[...]
    wt3 = jnp.concatenate([wt_ref[...], b_ref[...]], axis=0)
    C = B // 2
    copies = []
    for q in range(_NR):
        for h in range(2):
            scratch[q * R:(q + 1) * R, h * C:(h + 1) * C] = jax.lax.dot_general(
                wt3[:, q * R:(q + 1) * R], m[:, h * C:(h + 1) * C],
                dimension_numbers=(((0,), (0,)), ((), ())),
                preferred_element_type=jnp.float32,
            )
            cp = pltpu.make_async_copy(
                scratch.at[pl.ds(q * R, R), pl.ds(h * C, C)],
                o_hbm.at[pl.ds(q * R, R), pl.ds(h * C, C)],
                sems.at[q * 2 + h],
            )
            cp.start()
            copies.append(cp)
    for cp in copies:
        cp.wait()


@jax.jit
def _run(xt, emb2, wt2, brow):
    B = xt.shape[1]
    N = wt2.shape[1]
    return pl.pallas_call(
        _body,
        in_specs=[
            pl.BlockSpec(memory_space=pltpu.MemorySpace.VMEM),
            pl.BlockSpec(memory_space=pltpu.MemorySpace.VMEM),
            pl.BlockSpec(memory_space=pltpu.MemorySpace.VMEM),
            pl.BlockSpec(memory_space=pltpu.MemorySpace.VMEM),
        ],
        out_specs=pl.BlockSpec(memory_space=pltpu.MemorySpace.HBM),
        out_shape=jax.ShapeDtypeStruct((N, B), jnp.float32),
        scratch_shapes=[
            pltpu.VMEM((3, B), jnp.float32),
            pltpu.VMEM((N, B), jnp.float32),
            pltpu.SemaphoreType.DMA((_NR * 2,)),
        ],
    )(xt, emb2, wt2, brow)


def kernel(x, emb16, fc1_w, fc1_b):
    N = fc1_w.shape[0]
    xt = x.T
    emb2 = emb16.reshape(1, 2)
    wt2 = fc1_w.T
    brow = fc1_b.reshape(1, N)
    out_t = _run(xt, emb2, wt2, brow)
    return out_t.T


# speedup vs baseline: 1.0084x; 1.0084x over previous
import jax
import jax.numpy as jnp
from jax.experimental import pallas as pl
from jax.experimental.pallas import tpu as pltpu

_NR = 4


def _body(xt_ref, emb_ref, wt_ref, b_ref, o_hbm, mscr, scratch, sems):
    B = xt_ref.shape[1]
    N = scratch.shape[0]
    R = N // _NR
    x0 = xt_ref[0:1, :]
    x1 = xt_ref[1:2, :]
    e0 = emb_ref[0, 0]
    e1 = emb_ref[0, 1]
    e = jnp.where(x1 >= 1.0, e1, e0)
    one = jnp.ones_like(x0)
    mscr[...] = jnp.concatenate([x0, e, one], axis=0)
    m = mscr[...]
    wt3 = jnp.concatenate([wt_ref[...], b_ref[...]], axis=0)
    copies = []
    for q in range(_NR):
        scratch[q * R:(q + 1) * R, :] = jax.lax.dot_general(
            wt3[:, q * R:(q + 1) * R], m,
            dimension_numbers=(((0,), (0,)), ((), ())),
            preferred_element_type=jnp.float32,
        )
        cp = pltpu.make_async_copy(
            scratch.at[pl.ds(q * R, R), :],
            o_hbm.at[pl.ds(q * R, R), :],
            sems.at[q],
        )
        cp.start()
        copies.append(cp)
    for cp in copies:
        cp.wait()


@jax.jit
def _run(xt, emb2, wt2, brow):
    B = xt.shape[1]
    N = wt2.shape[1]
    return pl.pallas_call(
        _body,
        in_specs=[
            pl.BlockSpec(memory_space=pltpu.MemorySpace.VMEM),
            pl.BlockSpec(memory_space=pltpu.MemorySpace.VMEM),
            pl.BlockSpec(memory_space=pltpu.MemorySpace.VMEM),
            pl.BlockSpec(memory_space=pltpu.MemorySpace.VMEM),
        ],
        out_specs=pl.BlockSpec(memory_space=pltpu.MemorySpace.HBM),
        out_shape=jax.ShapeDtypeStruct((N, B), jnp.float32),
        scratch_shapes=[
            pltpu.VMEM((3, B), jnp.float32),
            pltpu.VMEM((N, B), jnp.float32),
            pltpu.SemaphoreType.DMA((_NR,)),
        ],
    )(xt, emb2, wt2, brow)


def kernel(x, emb16, fc1_w, fc1_b):
    N = fc1_w.shape[0]
    xt = x.T
    emb2 = emb16.reshape(1, 2)
    wt2 = fc1_w.T
    brow = fc1_b.reshape(1, N)
    out_t = _run(xt, emb2, wt2, brow)
    return out_t.T


# speedup vs baseline: 1.0205x; 1.0120x over previous
import jax
import jax.numpy as jnp
from jax.experimental import pallas as pl
from jax.experimental.pallas import tpu as pltpu

_NQ = 8


def _body(xt_ref, emb_ref, wt_ref, b_ref, o_hbm, scratch, sems):
    B = xt_ref.shape[1]
    N = scratch.shape[0]
    C = B // _NQ
    e0 = emb_ref[0, 0]
    e1 = emb_ref[0, 1]
    wt3 = jnp.concatenate([wt_ref[...], b_ref[...]], axis=0)
    copies = []
    for q in range(_NQ):
        x0 = xt_ref[0:1, pl.ds(q * C, C)]
        x1 = xt_ref[1:2, pl.ds(q * C, C)]
        e = jnp.where(x1 >= 1.0, e1, e0)
        one = jnp.ones_like(x0)
        m = jnp.concatenate([x0, e, one], axis=0)
        scratch[:, q * C:(q + 1) * C] = jax.lax.dot_general(
            wt3, m,
            dimension_numbers=(((0,), (0,)), ((), ())),
            preferred_element_type=jnp.float32,
        )
        cp = pltpu.make_async_copy(
            scratch.at[:, pl.ds(q * C, C)],
            o_hbm.at[:, pl.ds(q * C, C)],
            sems.at[q],
        )
        cp.start()
        copies.append(cp)
    for cp in copies:
        cp.wait()


@jax.jit
def _run(xt, emb2, wt2, brow):
    B = xt.shape[1]
    N = wt2.shape[1]
    return pl.pallas_call(
        _body,
        in_specs=[
            pl.BlockSpec(memory_space=pltpu.MemorySpace.VMEM),
            pl.BlockSpec(memory_space=pltpu.MemorySpace.VMEM),
            pl.BlockSpec(memory_space=pltpu.MemorySpace.VMEM),
            pl.BlockSpec(memory_space=pltpu.MemorySpace.VMEM),
        ],
        out_specs=pl.BlockSpec(memory_space=pltpu.MemorySpace.HBM),
        out_shape=jax.ShapeDtypeStruct((N, B), jnp.float32),
        scratch_shapes=[
            pltpu.VMEM((N, B), jnp.float32),
            pltpu.SemaphoreType.DMA((_NQ,)),
        ],
    )(xt, emb2, wt2, brow)


def kernel(x, emb16, fc1_w, fc1_b):
    N = fc1_w.shape[0]
    xt = x.T
    emb2 = emb16.reshape(1, 2)
    wt2 = fc1_w.T
    brow = fc1_b.reshape(1, N)
    out_t = _run(xt, emb2, wt2, brow)
    return out_t.T
